# baseline (speedup 1.0000x reference)
import functools

import jax
import jax.numpy as jnp
from jax import lax
from jax.experimental import pallas as pl
from jax.experimental.pallas import tpu as pltpu, tpu_sc as plsc

N_FIELDS = 26
VOCAB = 100000
EMBED = 32
BATCH = 16384
PACK = 4

_INFO = plsc.get_sparse_core_info()
_NC, _NS = _INFO.num_cores, _INFO.num_subcores
_NW = _NC * _NS
_BPW = BATCH // _NW
_NG = 4
_GSZ = _BPW // _NG


def _body(ids_hbm, tab_hbm, out_hbm, idx_v, off_v, rowsA, rowsB, t0, t1,
          isem, gsem, wsem0, wsem1):
    wid = lax.axis_index("s") * _NC + lax.axis_index("c")
    base = wid * _BPW

    def fire_ids(f, _):
        for g in range(_NG):
            pltpu.async_copy(
                ids_hbm.at[pl.ds(f * BATCH + base + g * _GSZ, _GSZ)],
                idx_v.at[f, g],
                isem,
            )
        return 0

    lax.fori_loop(0, N_FIELDS, fire_ids, 0)

    def drain_ids(t, _):
        pltpu.make_async_copy(
            ids_hbm.at[pl.ds(0, _GSZ)], idx_v.at[0, 0], isem
        ).wait()
        return 0

    lax.fori_loop(0, N_FIELDS * _NG, drain_ids, 0)

    def add_off(t, _):
        f = t // _NG
        g = t - f * _NG
        for u in range(_GSZ // 16):
            s16 = pl.ds(u * 16, 16)
            flat = idx_v[f, g, s16] + f * VOCAB
            idx_v[f, g, s16] = flat >> 2
            off_v[f, g, s16] = (flat & 3) * EMBED
        return 0

    lax.fori_loop(0, N_FIELDS * _NG, add_off, 0)

    lanes = lax.iota(jnp.int32, 16)

    def select_group(f, g, rows_v, t_v):
        def sel16(k, _):
            b0 = k * 16
            b_vec = lanes + b0
            o_vec = off_v[f, g, pl.ds(b0, 16)]
            for e in range(EMBED):
                vals = plsc.load_gather(rows_v, [b_vec, o_vec + e])
                t_v[e, pl.ds(g * _GSZ + b0, 16)] = vals
            return 0

        lax.fori_loop(0, _GSZ // 16, sel16, 0)

    def fire_gather(f, g, rows_v):
        return pltpu.async_copy(
            tab_hbm.at[idx_v.at[f, g]], rows_v, gsem
        )

    def do_field(f, t_v, wsem, first):
        @pl.when(jnp.logical_not(first))
        def _():
            pltpu.make_async_copy(
                t_v, out_hbm.at[pl.ds(0, EMBED), pl.ds(base, _BPW)], wsem
            ).wait()

        c0 = fire_gather(f, 0, rowsA)
        c1 = fire_gather(f, 1, rowsB)
        c0.wait()
        select_group(f, 0, rowsA, t_v)
        c2 = fire_gather(f, 2, rowsA)
        c1.wait()
        select_group(f, 1, rowsB, t_v)
        c3 = fire_gather(f, 3, rowsB)
        c2.wait()
        select_group(f, 2, rowsA, t_v)
        c3.wait()
        select_group(f, 3, rowsB, t_v)

        pltpu.async_copy(
            t_v, out_hbm.at[pl.ds(f * EMBED, EMBED), pl.ds(base, _BPW)], wsem
        )

    def pair(p, _):
        do_field(2 * p, t0, wsem0, p == 0)
        do_field(2 * p + 1, t1, wsem1, p == 0)
        return 0

    lax.fori_loop(0, N_FIELDS // 2, pair, 0)

    pltpu.make_async_copy(
        t0, out_hbm.at[pl.ds(0, EMBED), pl.ds(base, _BPW)], wsem0
    ).wait()
    pltpu.make_async_copy(
        t1, out_hbm.at[pl.ds(0, EMBED), pl.ds(base, _BPW)], wsem1
    ).wait()


@functools.partial(jax.jit, static_argnums=())
def kernel(ids, tables):
    ids_flat = ids.reshape(N_FIELDS * BATCH)
    tab = tables.reshape(N_FIELDS * VOCAB // PACK, PACK * EMBED)
    run = pl.kernel(
        _body,
        out_type=jax.ShapeDtypeStruct((N_FIELDS * EMBED, BATCH), jnp.float32),
        mesh=plsc.VectorSubcoreMesh(core_axis_name="c", subcore_axis_name="s"),
        scratch_types=[
            pltpu.VMEM((N_FIELDS, _NG, _GSZ), jnp.int32),
            pltpu.VMEM((N_FIELDS, _NG, _GSZ), jnp.int32),
            pltpu.VMEM((_GSZ, PACK * EMBED), jnp.float32),
            pltpu.VMEM((_GSZ, PACK * EMBED), jnp.float32),
            pltpu.VMEM((EMBED, _BPW), jnp.float32),
            pltpu.VMEM((EMBED, _BPW), jnp.float32),
            pltpu.SemaphoreType.DMA,
            pltpu.SemaphoreType.DMA,
            pltpu.SemaphoreType.DMA,
            pltpu.SemaphoreType.DMA,
        ],
        compiler_params=pltpu.CompilerParams(needs_layout_passes=False),
    )
    return run(ids_flat, tab).T


# speedup vs baseline: 1.4650x; 1.4650x over previous
import functools

import jax
import jax.numpy as jnp
from jax import lax
from jax.experimental import pallas as pl
from jax.experimental.pallas import tpu as pltpu, tpu_sc as plsc

N_FIELDS = 26
VOCAB = 100000
EMBED = 32
BATCH = 16384

_INFO = plsc.get_sparse_core_info()
_NC, _NS = _INFO.num_cores, _INFO.num_subcores
_NW = _NC * _NS
_BPW = BATCH // _NW
_NG = 4
_GSZ = _BPW // _NG


def _body(ids_hbm, tab_hbm, out_hbm, idx_v, e0_v, e1_v, t0, t1,
          isem, gsem, wsem0, wsem1):
    wid = lax.axis_index("s") * _NC + lax.axis_index("c")
    base = wid * _BPW

    def fire_ids(f, _):
        for g in range(_NG):
            pltpu.async_copy(
                ids_hbm.at[pl.ds(f * BATCH + base + g * _GSZ, _GSZ)],
                idx_v.at[f, g],
                isem,
            )
        return 0

    lax.fori_loop(0, N_FIELDS, fire_ids, 0)

    def drain_ids(t, _):
        pltpu.make_async_copy(
            ids_hbm.at[pl.ds(0, _GSZ)], idx_v.at[0, 0], isem
        ).wait()
        return 0

    lax.fori_loop(0, N_FIELDS * _NG, drain_ids, 0)

    def build_eidx(f, e_v):
        def per_eg(t, _):
            e = t // _NG
            g = t - e * _NG
            off = (f * EMBED + e) * VOCAB
            for u in range(_GSZ // 16):
                s16 = pl.ds(u * 16, 16)
                e_v[e, g, s16] = idx_v[f, g, s16] + off
            return 0

        lax.fori_loop(0, EMBED * _NG, per_eg, 0)

    def fire_field(e_v, t_v):
        def fire_e(e, _):
            for g in range(_NG):
                pltpu.async_copy(
                    tab_hbm.at[e_v.at[e, g]],
                    t_v.at[e, pl.ds(g * _GSZ, _GSZ)],
                    gsem,
                )
            return 0

        lax.fori_loop(0, EMBED, fire_e, 0)

    def drain_field():
        def drain_e(t, _):
            pltpu.make_async_copy(
                tab_hbm.at[e0_v.at[0, 0]],
                t0.at[0, pl.ds(0, _GSZ)],
                gsem,
            ).wait()
            return 0

        lax.fori_loop(0, EMBED * _NG, drain_e, 0)

    def reclaim(t_v, wsem):
        pltpu.make_async_copy(
            t_v, out_hbm.at[pl.ds(0, EMBED), pl.ds(base, _BPW)], wsem
        ).wait()

    def write_field(f, t_v, wsem):
        pltpu.async_copy(
            t_v, out_hbm.at[pl.ds(f * EMBED, EMBED), pl.ds(base, _BPW)], wsem
        )

    build_eidx(0, e0_v)

    def pair(p, _):
        f0 = 2 * p

        @pl.when(p > 0)
        def _():
            reclaim(t0, wsem0)

        fire_field(e0_v, t0)
        build_eidx(f0 + 1, e1_v)
        drain_field()
        write_field(f0, t0, wsem0)

        @pl.when(p > 0)
        def _():
            reclaim(t1, wsem1)

        fire_field(e1_v, t1)

        @pl.when(p < N_FIELDS // 2 - 1)
        def _():
            build_eidx(f0 + 2, e0_v)

        drain_field()
        write_field(f0 + 1, t1, wsem1)
        return 0

    lax.fori_loop(0, N_FIELDS // 2, pair, 0)

    reclaim(t0, wsem0)
    reclaim(t1, wsem1)


@functools.partial(jax.jit, static_argnums=())
def kernel(ids, tables):
    ids_flat = ids.reshape(N_FIELDS * BATCH)
    tab = tables.transpose(0, 2, 1).reshape(N_FIELDS * EMBED * VOCAB)
    run = pl.kernel(
        _body,
        out_type=jax.ShapeDtypeStruct((N_FIELDS * EMBED, BATCH), jnp.float32),
        mesh=plsc.VectorSubcoreMesh(core_axis_name="c", subcore_axis_name="s"),
        scratch_types=[
            pltpu.VMEM((N_FIELDS, _NG, _GSZ), jnp.int32),
            pltpu.VMEM((EMBED, _NG, _GSZ), jnp.int32),
            pltpu.VMEM((EMBED, _NG, _GSZ), jnp.int32),
            pltpu.VMEM((EMBED, _BPW), jnp.float32),
            pltpu.VMEM((EMBED, _BPW), jnp.float32),
            pltpu.SemaphoreType.DMA,
            pltpu.SemaphoreType.DMA,
            pltpu.SemaphoreType.DMA,
            pltpu.SemaphoreType.DMA,
        ],
        compiler_params=pltpu.CompilerParams(use_tc_tiling_on_sc=False),
    )
    return run(ids_flat, tab).T


# speedup vs baseline: 1.4702x; 1.0036x over previous
import functools

import jax
import jax.numpy as jnp
from jax import lax
from jax.experimental import pallas as pl
from jax.experimental.pallas import tpu as pltpu, tpu_sc as plsc

N_FIELDS = 26
VOCAB = 100000
EMBED = 32
BATCH = 16384

_INFO = plsc.get_sparse_core_info()
_NC, _NS = _INFO.num_cores, _INFO.num_subcores
_NW = _NC * _NS
_BPW = BATCH // _NW
_NG = 4
_GSZ = _BPW // _NG


def _body(ids_hbm, tab_hbm, out_hbm, idx_v, e0_v, e1_v, t0, t1,
          isem, gsem, wsem0, wsem1):
    wid = lax.axis_index("s") * _NC + lax.axis_index("c")
    base = wid * _BPW

    def fire_ids(f, _):
        for g in range(_NG):
            pltpu.async_copy(
                ids_hbm.at[pl.ds(f * BATCH + base + g * _GSZ, _GSZ)],
                idx_v.at[f, g],
                isem,
            )
        return 0

    lax.fori_loop(0, N_FIELDS, fire_ids, 0)

    def drain_ids(t, _):
        pltpu.make_async_copy(
            ids_hbm.at[pl.ds(0, _GSZ)], idx_v.at[0, 0], isem
        ).wait()
        return 0

    lax.fori_loop(0, N_FIELDS * _NG, drain_ids, 0)

    def build_eidx(f, e_v):
        def per_eg(t, _):
            e = t // _NG
            g = t - e * _NG
            off = (f * EMBED + e) * VOCAB
            for u in range(_GSZ // 16):
                s16 = pl.ds(u * 16, 16)
                e_v[e, g, s16] = idx_v[f, g, s16] + off
            return 0

        lax.fori_loop(0, EMBED * _NG, per_eg, 0)

    def fire_field(e_v, t_v):
        def fire_e(e, _):
            for g in range(_NG):
                pltpu.async_copy(
                    tab_hbm.at[e_v.at[e, g]],
                    t_v.at[e, pl.ds(g * _GSZ, _GSZ)],
                    gsem,
                )
            return 0

        lax.fori_loop(0, EMBED, fire_e, 0)

    def drain_field():
        pltpu.make_async_copy(
            out_hbm.at[pl.ds(0, EMBED), pl.ds(0, _BPW)], t0, gsem
        ).wait()

    def reclaim(t_v, wsem):
        pltpu.make_async_copy(
            t_v, out_hbm.at[pl.ds(0, EMBED), pl.ds(base, _BPW)], wsem
        ).wait()

    def write_field(f, t_v, wsem):
        pltpu.async_copy(
            t_v, out_hbm.at[pl.ds(f * EMBED, EMBED), pl.ds(base, _BPW)], wsem
        )

    build_eidx(0, e0_v)

    def pair(p, _):
        f0 = 2 * p

        @pl.when(p > 0)
        def _():
            reclaim(t0, wsem0)

        fire_field(e0_v, t0)
        build_eidx(f0 + 1, e1_v)
        drain_field()
        write_field(f0, t0, wsem0)

        @pl.when(p > 0)
        def _():
            reclaim(t1, wsem1)

        fire_field(e1_v, t1)

        @pl.when(p < N_FIELDS // 2 - 1)
        def _():
            build_eidx(f0 + 2, e0_v)

        drain_field()
        write_field(f0 + 1, t1, wsem1)
        return 0

    lax.fori_loop(0, N_FIELDS // 2, pair, 0)

    reclaim(t0, wsem0)
    reclaim(t1, wsem1)


@functools.partial(jax.jit, static_argnums=())
def kernel(ids, tables):
    ids_flat = ids.reshape(N_FIELDS * BATCH)
    tab = tables.transpose(0, 2, 1).reshape(N_FIELDS * EMBED * VOCAB)
    run = pl.kernel(
        _body,
        out_type=jax.ShapeDtypeStruct((N_FIELDS * EMBED, BATCH), jnp.float32),
        mesh=plsc.VectorSubcoreMesh(core_axis_name="c", subcore_axis_name="s"),
        scratch_types=[
            pltpu.VMEM((N_FIELDS, _NG, _GSZ), jnp.int32),
            pltpu.VMEM((EMBED, _NG, _GSZ), jnp.int32),
            pltpu.VMEM((EMBED, _NG, _GSZ), jnp.int32),
            pltpu.VMEM((EMBED, _BPW), jnp.float32),
            pltpu.VMEM((EMBED, _BPW), jnp.float32),
            pltpu.SemaphoreType.DMA,
            pltpu.SemaphoreType.DMA,
            pltpu.SemaphoreType.DMA,
            pltpu.SemaphoreType.DMA,
        ],
        compiler_params=pltpu.CompilerParams(use_tc_tiling_on_sc=False),
    )
    return run(ids_flat, tab).T


# speedup vs baseline: 1.4931x; 1.0156x over previous
import functools

import jax
import jax.numpy as jnp
from jax import lax
from jax.experimental import pallas as pl
from jax.experimental.pallas import tpu as pltpu, tpu_sc as plsc

N_FIELDS = 26
VOCAB = 100000
EMBED = 32
BATCH = 16384

_INFO = plsc.get_sparse_core_info()
_NC, _NS = _INFO.num_cores, _INFO.num_subcores
_NW = _NC * _NS
_BPW = BATCH // _NW
_NG = 4
_GSZ = _BPW // _NG


def _body(ids_hbm, tab_hbm, out_hbm, idx_v, e0_v, e1_v, t0, t1, t2, t3,
          isem, gsem0, gsem1, wsem0, wsem1, wsem2, wsem3):
    wid = lax.axis_index("s") * _NC + lax.axis_index("c")
    base = wid * _BPW

    def fire_ids(f, _):
        for g in range(_NG):
            pltpu.async_copy(
                ids_hbm.at[pl.ds(f * BATCH + base + g * _GSZ, _GSZ)],
                idx_v.at[f, g],
                isem,
            )
        return 0

    lax.fori_loop(0, N_FIELDS, fire_ids, 0)

    def drain_ids(t, _):
        pltpu.make_async_copy(
            ids_hbm.at[pl.ds(0, _GSZ)], idx_v.at[0, 0], isem
        ).wait()
        return 0

    lax.fori_loop(0, N_FIELDS * _NG, drain_ids, 0)

    def build_eidx(f, e_v):
        def per_eg(t, _):
            e = t // _NG
            g = t - e * _NG
            off = (f * EMBED + e) * VOCAB
            for u in range(_GSZ // 16):
                s16 = pl.ds(u * 16, 16)
                e_v[e, g, s16] = idx_v[f, g, s16] + off
            return 0

        lax.fori_loop(0, EMBED * _NG, per_eg, 0)

    def fire_field(e_v, t_v, gsem):
        def fire_e(e, _):
            for g in range(_NG):
                pltpu.async_copy(
                    tab_hbm.at[e_v.at[e, g]],
                    t_v.at[e, pl.ds(g * _GSZ, _GSZ)],
                    gsem,
                )
            return 0

        lax.fori_loop(0, EMBED, fire_e, 0)

    def drain_field(gsem):
        pltpu.make_async_copy(
            out_hbm.at[pl.ds(0, EMBED), pl.ds(0, _BPW)], t0, gsem
        ).wait()

    def reclaim(t_v, wsem):
        pltpu.make_async_copy(
            t_v, out_hbm.at[pl.ds(0, EMBED), pl.ds(base, _BPW)], wsem
        ).wait()

    def write_field(f, t_v, wsem):
        pltpu.async_copy(
            t_v, out_hbm.at[pl.ds(f * EMBED, EMBED), pl.ds(base, _BPW)], wsem
        )

    ts = (t0, t1, t2, t3)
    ws = (wsem0, wsem1, wsem2, wsem3)
    gs = (gsem0, gsem1)
    es = (e0_v, e1_v)

    build_eidx(0, e0_v)
    fire_field(e0_v, t0, gsem0)
    build_eidx(1, e1_v)
    fire_field(e1_v, t1, gsem1)

    def quad(q, _):
        f_base = 4 * q
        for k in range(4):
            f = f_base + k
            drain_field(gs[k % 2])

            @pl.when(f + 2 < N_FIELDS)
            def _():
                build_eidx(f + 2, es[k % 2])

                @pl.when(f >= 2)
                def _():
                    reclaim(ts[(k + 2) % 4], ws[(k + 2) % 4])

                fire_field(es[k % 2], ts[(k + 2) % 4], gs[k % 2])

            write_field(f, ts[k], ws[k])
        return 0

    lax.fori_loop(0, N_FIELDS // 4, quad, 0)

    drain_field(gsem0)
    write_field(N_FIELDS - 2, t0, wsem0)
    drain_field(gsem1)
    write_field(N_FIELDS - 1, t1, wsem1)

    for k in range(4):
        reclaim(ts[k], ws[k])


@functools.partial(jax.jit, static_argnums=())
def kernel(ids, tables):
    ids_flat = ids.reshape(N_FIELDS * BATCH)
    tab = tables.transpose(0, 2, 1).reshape(N_FIELDS * EMBED * VOCAB)
    run = pl.kernel(
        _body,
        out_type=jax.ShapeDtypeStruct((N_FIELDS * EMBED, BATCH), jnp.float32),
        mesh=plsc.VectorSubcoreMesh(core_axis_name="c", subcore_axis_name="s"),
        scratch_types=[
            pltpu.VMEM((N_FIELDS, _NG, _GSZ), jnp.int32),
            pltpu.VMEM((EMBED, _NG, _GSZ), jnp.int32),
            pltpu.VMEM((EMBED, _NG, _GSZ), jnp.int32),
            pltpu.VMEM((EMBED, _BPW), jnp.float32),
            pltpu.VMEM((EMBED, _BPW), jnp.float32),
            pltpu.VMEM((EMBED, _BPW), jnp.float32),
            pltpu.VMEM((EMBED, _BPW), jnp.float32),
            pltpu.SemaphoreType.DMA,
            pltpu.SemaphoreType.DMA,
            pltpu.SemaphoreType.DMA,
            pltpu.SemaphoreType.DMA,
            pltpu.SemaphoreType.DMA,
            pltpu.SemaphoreType.DMA,
            pltpu.SemaphoreType.DMA,
        ],
        compiler_params=pltpu.CompilerParams(use_tc_tiling_on_sc=False),
    )
    return run(ids_flat, tab).T
